# baseline (speedup 1.0000x reference)
import jax
import jax.numpy as jnp
from jax import lax
from jax.experimental import pallas as pl
from jax.experimental.pallas import tpu as pltpu
from jax.experimental.pallas import tpu_sc as plsc

NUM_HEADS = 16
MAX_DIST = 128
S = 2048
LANES = 16
NCLS = 50
CPAD = 576
ROWT = S // 8
COLT = S // 128
LMAX = 14
RMAX = 13


def _rpb_body(cols_hbm, out_hbm, col_v, tiles_v, lconst_v, rconst_v, sem):
    c = lax.axis_index("c")
    s = lax.axis_index("s")
    h = s

    pltpu.sync_copy(cols_hbm.at[pl.ds(h * CPAD, CPAD)], col_v)

    def build_body(n, carry):
        cls = n // 64 - 32
        i = (n // 8) % 8
        jj = n % 8
        vals = col_v[pl.ds(288 + 8 * cls - i + jj * LANES, LANES)]
        tiles_v[n // 64, i, pl.ds(jj * LANES, LANES)] = vals
        return carry

    lax.fori_loop(0, NCLS * 64, build_body, 0)

    left = col_v[pl.ds(0, LANES)]
    right = col_v[pl.ds(CPAD - LANES, LANES)]

    def lfill_body(n, carry):
        lconst_v[n // (LMAX * 8), pl.ds((n % (LMAX * 8)) * LANES, LANES)] = left
        return carry

    def rfill_body(n, carry):
        rconst_v[n // (RMAX * 8), pl.ds((n % (RMAX * 8)) * LANES, LANES)] = right
        return carry

    lax.fori_loop(0, 8 * LMAX * 8, lfill_body, 0)
    lax.fori_loop(0, 8 * RMAX * 8, rfill_body, 0)

    for cc in range(2):

        @pl.when(c == cc)
        def _issue(cc=cc):
            a0 = 128 * cc

            def drain64():
                pltpu.make_async_copy(
                    out_hbm.at[0, pl.ds(0, 16), pl.ds(0, 8), pl.ds(0, 128)],
                    tiles_v.at[pl.ds(0, 16)], sem
                ).wait()

            for lg in range(8):
                lrun = 8 * cc + lg - 1

                def a_body(t, carry, lg=lg, lrun=lrun):
                    a = a0 + 16 * lg + t
                    row = pl.multiple_of(a * 8, 8)
                    for bb in range(max(lrun, 0)):
                        pltpu.async_copy(
                            tiles_v.at[0],
                            out_hbm.at[0, h, pl.ds(row, 8),
                                       pl.ds(128 * bb, 128)],
                            sem)
                    for k in range(3):
                        bb = lrun + k
                        if 0 <= bb <= 15:
                            pltpu.async_copy(
                                tiles_v.at[16 * k + 16 - t],
                                out_hbm.at[0, h, pl.ds(row, 8),
                                           pl.ds(128 * bb, 128)],
                                sem)
                    for bb in range(lrun + 3, COLT):
                        pltpu.async_copy(
                            tiles_v.at[NCLS - 1],
                            out_hbm.at[0, h, pl.ds(row, 8),
                                       pl.ds(128 * bb, 128)],
                            sem)
                    if lg > 0:
                        drain64()
                    return carry

                lax.fori_loop(0, 16, a_body, 0)

            def tail_body(t, carry):
                drain64()
                return carry

            lax.fori_loop(0, 16, tail_body, 0)


def kernel(seq_length, table):
    del seq_length
    cols = jnp.pad(table.T, ((0, 0), (160, CPAD - 160 - (2 * MAX_DIST + 1))),
                   mode="edge").reshape(-1)
    mesh = plsc.VectorSubcoreMesh(core_axis_name="c", subcore_axis_name="s")
    f = pl.kernel(
        _rpb_body,
        mesh=mesh,
        out_type=jax.ShapeDtypeStruct((1, NUM_HEADS, S, S), jnp.float32),
        scratch_types=[
            pltpu.VMEM((CPAD,), jnp.float32),
            pltpu.VMEM((NCLS, 8, 128), jnp.float32),
            pltpu.VMEM((8, 128 * LMAX), jnp.float32),
            pltpu.VMEM((8, 128 * RMAX), jnp.float32),
            pltpu.SemaphoreType.DMA,
        ],
        compiler_params=pltpu.CompilerParams(use_tc_tiling_on_sc=True),
    )
    return f(cols)


# speedup vs baseline: 1.1075x; 1.1075x over previous
import jax
import jax.numpy as jnp
from jax import lax
from jax.experimental import pallas as pl
from jax.experimental.pallas import tpu as pltpu
from jax.experimental.pallas import tpu_sc as plsc

NUM_HEADS = 16
MAX_DIST = 128
S = 2048
LANES = 16
NCLS = 50
CPAD = 576
ROWT = S // 8
COLT = S // 128
INFLIGHT = 32


def _rpb_body(cols_hbm, out_hbm, col_v, tiles_v, sem):
    c = lax.axis_index("c")
    s = lax.axis_index("s")
    h = s

    pltpu.sync_copy(cols_hbm.at[pl.ds(h * CPAD, CPAD)], col_v)

    def build_body(n, carry):
        cls = n // 64 - 32
        i = (n // 8) % 8
        jj = n % 8
        vals = col_v[pl.ds(288 + 8 * cls - i + jj * LANES, LANES)]
        tiles_v[n // 64, i, pl.ds(jj * LANES, LANES)] = vals
        return carry

    lax.fori_loop(0, NCLS * 64, build_body, 0)

    a_base = c * (ROWT // 2)
    n_tiles = (ROWT // 2) * COLT

    def issue(n):
        b = n % COLT
        a = a_base + n // COLT
        cls_idx = jnp.clip(16 * b - a, -32, 17) + 32
        pltpu.async_copy(
            tiles_v.at[cls_idx],
            out_hbm.at[0, h,
                       pl.ds(pl.multiple_of(a * 8, 8), 8),
                       pl.ds(pl.multiple_of(b * 128, 128), 128)],
            sem,
        )

    def drain():
        pltpu.make_async_copy(
            out_hbm.at[0, 0, pl.ds(0, 8), pl.ds(0, 128)], tiles_v.at[0], sem
        ).wait()

    def prime_body(n, carry):
        issue(n)
        return carry

    def steady_body(n, carry):
        issue(n)
        drain()
        return carry

    def tail_body(n, carry):
        drain()
        return carry

    lax.fori_loop(0, INFLIGHT, prime_body, 0)
    lax.fori_loop(INFLIGHT, n_tiles, steady_body, 0)
    lax.fori_loop(0, INFLIGHT, tail_body, 0)


def kernel(seq_length, table):
    del seq_length
    cols = jnp.pad(table.T, ((0, 0), (160, CPAD - 160 - (2 * MAX_DIST + 1))),
                   mode="edge").reshape(-1)
    mesh = plsc.VectorSubcoreMesh(core_axis_name="c", subcore_axis_name="s")
    f = pl.kernel(
        _rpb_body,
        mesh=mesh,
        out_type=jax.ShapeDtypeStruct((1, NUM_HEADS, S, S), jnp.float32),
        scratch_types=[
            pltpu.VMEM((CPAD,), jnp.float32),
            pltpu.VMEM((NCLS, 8, 128), jnp.float32),
            pltpu.SemaphoreType.DMA,
        ],
        compiler_params=pltpu.CompilerParams(use_tc_tiling_on_sc=True),
    )
    return f(cols)


# speedup vs baseline: 1.1233x; 1.0143x over previous
import jax
import jax.numpy as jnp
from jax import lax
from jax.experimental import pallas as pl
from jax.experimental.pallas import tpu as pltpu
from jax.experimental.pallas import tpu_sc as plsc

NUM_HEADS = 16
MAX_DIST = 128
S = 2048
LANES = 16
NCLS = 50
CPAD = 576
ROWT = S // 8
COLT = S // 128
INFLIGHT = 32


def _rpb_body(cols_hbm, out_hbm, col_v, tiles_v, sem):
    c = lax.axis_index("c")
    s = lax.axis_index("s")
    h = s

    pltpu.sync_copy(cols_hbm.at[pl.ds(h * CPAD, CPAD)], col_v)

    def build_body(n, carry):
        cls = n // 64 - 32
        i = (n // 8) % 8
        jj = n % 8
        vals = col_v[pl.ds(288 + 8 * cls - i + jj * LANES, LANES)]
        tiles_v[n // 64, i, pl.ds(jj * LANES, LANES)] = vals
        return carry

    lax.fori_loop(0, NCLS * 64, build_body, 0)

    a_base = c * (ROWT // 2)
    n_tiles = (ROWT // 2) * COLT

    def issue(n):
        b = n % COLT
        a = c + 2 * (n // COLT)
        cls_idx = jnp.clip(16 * b - a, -32, 17) + 32
        pltpu.async_copy(
            tiles_v.at[cls_idx],
            out_hbm.at[0, h,
                       pl.ds(pl.multiple_of(a * 8, 8), 8),
                       pl.ds(pl.multiple_of(b * 128, 128), 128)],
            sem,
        )

    def drain():
        pltpu.make_async_copy(
            out_hbm.at[0, 0, pl.ds(0, 8), pl.ds(0, 128)], tiles_v.at[0], sem
        ).wait()

    def prime_body(n, carry):
        issue(n)
        return carry

    def steady_body(n, carry):
        issue(n)
        drain()
        return carry

    def tail_body(n, carry):
        drain()
        return carry

    lax.fori_loop(0, INFLIGHT, prime_body, 0)
    lax.fori_loop(INFLIGHT, n_tiles, steady_body, 0)
    lax.fori_loop(0, INFLIGHT, tail_body, 0)


def kernel(seq_length, table):
    del seq_length
    cols = jnp.pad(table.T, ((0, 0), (160, CPAD - 160 - (2 * MAX_DIST + 1))),
                   mode="edge").reshape(-1)
    mesh = plsc.VectorSubcoreMesh(core_axis_name="c", subcore_axis_name="s")
    f = pl.kernel(
        _rpb_body,
        mesh=mesh,
        out_type=jax.ShapeDtypeStruct((1, NUM_HEADS, S, S), jnp.float32),
        scratch_types=[
            pltpu.VMEM((CPAD,), jnp.float32),
            pltpu.VMEM((NCLS, 8, 128), jnp.float32),
            pltpu.SemaphoreType.DMA,
        ],
        compiler_params=pltpu.CompilerParams(use_tc_tiling_on_sc=True),
    )
    return f(cols)
